# speedup vs baseline: 1.4811x; 1.4811x over previous
import functools

import jax
import jax.numpy as jnp
from jax import lax
from jax.experimental import pallas as pl
from jax.experimental.pallas import tpu as pltpu
from jax.experimental.pallas import tpu_sc as plsc

_NUM_SAMPLES = 100000
_BATCH = 16384
_L = 16
_NC = 2
_NS = 16
_NW = _NC * _NS
_B_PER_W = _BATCH // _NW
_CHUNK = 128
_NCHUNK = _B_PER_W // _CHUNK

_mesh = plsc.VectorSubcoreMesh(core_axis_name="c", subcore_axis_name="s")


@functools.partial(
    pl.kernel,
    mesh=_mesh,
    out_type=jax.ShapeDtypeStruct((_NW, _L), jnp.float32),
    scratch_types=[
        pltpu.VMEM((_NCHUNK, _CHUNK), jnp.int32),
        pltpu.VMEM((_B_PER_W,), jnp.float32),
        pltpu.VMEM((_B_PER_W,), jnp.float32),
        pltpu.VMEM((_B_PER_W,), jnp.float32),
        pltpu.VMEM((_L,), jnp.float32),
        pltpu.VMEM((_L,), jnp.float32),
        pltpu.SemaphoreType.DMA,
    ],
)
def _sc_gather_sum(idx_hbm, a_hbm, b_hbm, c_hbm, m_hbm, out_hbm,
                   idx_v, av, bv, cv, mv, pv, sem):
    cid = lax.axis_index("c")
    sid = lax.axis_index("s")
    wid = sid * _NC + cid

    pltpu.sync_copy(idx_hbm.at[pl.ds(wid * _NCHUNK, _NCHUNK)], idx_v)
    pltpu.sync_copy(m_hbm, mv)

    copies = []
    for j in range(_NCHUNK):
        dst = pl.ds(j * _CHUNK, _CHUNK)
        copies.append(pltpu.async_copy(a_hbm.at[idx_v.at[j]], av.at[dst], sem))
        copies.append(pltpu.async_copy(b_hbm.at[idx_v.at[j]], bv.at[dst], sem))
        copies.append(pltpu.async_copy(c_hbm.at[idx_v.at[j]], cv.at[dst], sem))
    for cp in copies:
        cp.wait()

    acc_ac = jnp.zeros((_L,), jnp.float32)
    acc_b = jnp.zeros((_L,), jnp.float32)
    for i in range(_B_PER_W // _L):
        s = pl.ds(i * _L, _L)
        acc_ac = acc_ac + av[s] + cv[s]
        acc_b = acc_b + bv[s]

    pv[...] = acc_ac * mv[...] + acc_b
    pltpu.sync_copy(pv, out_hbm.at[wid])


def kernel(c_t_indices, M_prev, A, B, C):
    idx2d = c_t_indices.astype(jnp.int32).reshape(_NW * _NCHUNK, _CHUNK)
    m16 = jnp.full((_L,), M_prev, jnp.float32)
    partials = _sc_gather_sum(idx2d, A, B, C, m16)
    return jnp.sum(partials)
